# baseline (speedup 1.0000x reference)
import functools

import jax
import jax.numpy as jnp
from jax import lax
from jax.experimental import pallas as pl
from jax.experimental.pallas import tpu as pltpu
from jax.experimental.pallas import tpu_sc as plsc

VOCAB = 1000000
BATCH = 4096
SEQ = 200
DIM = 64
NUM_CORES = 2
NUM_SUBCORES = 16
NUM_WORKERS = NUM_CORES * NUM_SUBCORES
BPW = BATCH // NUM_WORKERS
CHUNK = 112
NCHUNK = 2
ROWS = NCHUNK * CHUNK
LANES = 16
NVEC = DIM // LANES

TRB = 4096
PAIR = 2 * TRB
NBLK = (VOCAB + PAIR - 1) // PAIR
PACK_ROWS = NBLK * TRB
SC_ROWS = 2 * PACK_ROWS


def _pack_body(x1_ref, x2_ref, o_ref):
    o_ref[...] = jnp.concatenate([x1_ref[...].T, x2_ref[...].T], axis=1)


def _to_row_major(table_t):
    return pl.pallas_call(
        _pack_body,
        grid=(NBLK,),
        in_specs=[
            pl.BlockSpec((DIM, TRB), lambda i: (0, 2 * i)),
            pl.BlockSpec(
                (DIM, TRB),
                lambda i: (0, jnp.minimum(2 * i + 1, VOCAB // TRB)),
            ),
        ],
        out_specs=pl.BlockSpec((TRB, 2 * DIM), lambda i: (i, 0)),
        out_shape=jax.ShapeDtypeStruct((PACK_ROWS, 2 * DIM), jnp.float32),
    )(table_t, table_t)


def _swem_body(idx_hbm, table_hbm, out_hbm, idx_v, rows_a, rows_b, out_v,
               sem_a, sem_b):
    wid = lax.axis_index("s") * NUM_CORES + lax.axis_index("c")
    base = wid * BPW

    pltpu.sync_copy(idx_hbm.at[pl.ds(base, BPW)], idx_v)

    def remap(e, _):
        for j in range(NCHUNK):
            for k in range(CHUNK // LANES):
                v = idx_v[e, j, pl.ds(k * LANES, LANES)]
                r = v & (PAIR - 1)
                adj = jnp.where(r >= TRB, PAIR - 1, 0)
                idx_v[e, j, pl.ds(k * LANES, LANES)] = v + r - adj
        return 0

    lax.fori_loop(0, BPW, remap, 0)

    def fire(e, rows_ref, sem):
        for j in range(NCHUNK):
            pltpu.async_copy(
                table_hbm.at[idx_v.at[e, j]],
                rows_ref.at[pl.ds(j * CHUNK, CHUNK)],
                sem,
            )

    def drain(rows_ref, sem):
        for j in range(NCHUNK):
            pltpu.make_async_copy(
                table_hbm.at[idx_v.at[0, j]],
                rows_ref.at[pl.ds(j * CHUNK, CHUNK)],
                sem,
            ).wait()

    def reduce_into(e, rows_ref):
        def body(r, accs):
            return tuple(
                accs[c] + rows_ref[r, pl.ds(c * LANES, LANES)]
                for c in range(NVEC)
            )

        zero = jnp.zeros((LANES,), jnp.float32)
        accs = lax.fori_loop(0, SEQ, body, (zero,) * NVEC, unroll=8)
        for c in range(NVEC):
            out_v[e, pl.ds(c * LANES, LANES)] = accs[c] * (1.0 / SEQ)

    fire(0, rows_a, sem_a)

    def outer(i, _):
        e = 2 * i
        fire(e + 1, rows_b, sem_b)
        drain(rows_a, sem_a)
        reduce_into(e, rows_a)

        @pl.when(e + 2 < BPW)
        def _():
            fire(e + 2, rows_a, sem_a)

        drain(rows_b, sem_b)
        reduce_into(e + 1, rows_b)
        return 0

    lax.fori_loop(0, BPW // 2, outer, 0)

    pltpu.sync_copy(out_v, out_hbm.at[pl.ds(base, BPW)])


def _swem_sc(idx, table):
    mesh = plsc.VectorSubcoreMesh(
        core_axis_name="c",
        subcore_axis_name="s",
        num_cores=NUM_CORES,
        num_subcores=NUM_SUBCORES,
    )
    k = pl.kernel(
        _swem_body,
        out_type=jax.ShapeDtypeStruct((BATCH, DIM), jnp.float32),
        mesh=mesh,
        scratch_types=[
            pltpu.VMEM((BPW, NCHUNK, CHUNK), jnp.int32),
            pltpu.VMEM((ROWS, DIM), jnp.float32),
            pltpu.VMEM((ROWS, DIM), jnp.float32),
            pltpu.VMEM((BPW, DIM), jnp.float32),
            pltpu.SemaphoreType.DMA,
            pltpu.SemaphoreType.DMA,
        ],
        compiler_params=pltpu.CompilerParams(use_tc_tiling_on_sc=False),
    )
    return k(idx, table)


@jax.jit
def _swem(input, table):
    packed = _to_row_major(table.T)
    table_sc = packed.reshape(SC_ROWS, DIM)
    a = input[:, :CHUNK]
    b = jnp.pad(input[:, CHUNK:], ((0, 0), (0, ROWS - SEQ)))
    idx3 = jnp.stack([a, b], axis=1)
    return _swem_sc(idx3, table_sc)


def kernel(input, table):
    return _swem(input, table)


# speedup vs baseline: 5.5558x; 5.5558x over previous
import functools

import jax
import jax.numpy as jnp
from jax import lax
from jax.experimental import pallas as pl
from jax.experimental.pallas import tpu as pltpu
from jax.experimental.pallas import tpu_sc as plsc

VOCAB = 1000000
BATCH = 4096
SEQ = 200
DIM = 64
NUM_CORES = 2
NUM_SUBCORES = 16
NUM_WORKERS = NUM_CORES * NUM_SUBCORES
BPW = BATCH // NUM_WORKERS
CHUNK = 96
TAIL = SEQ - 2 * CHUNK
NCHUNK = 3
CSIZES = (CHUNK, CHUNK, TAIL)
ROWS = SEQ
LANES = 16
NVEC = DIM // LANES

TRB = 4096
PAIR = 2 * TRB
NBLK = (VOCAB + PAIR - 1) // PAIR
PACK_ROWS = NBLK * TRB
SC_ROWS = 2 * PACK_ROWS


def _pack_body(x1_ref, x2_ref, o_ref):
    o_ref[...] = jnp.concatenate([x1_ref[...].T, x2_ref[...].T], axis=1)


def _to_row_major(table_t):
    return pl.pallas_call(
        _pack_body,
        grid=(NBLK,),
        in_specs=[
            pl.BlockSpec((DIM, TRB), lambda i: (0, 2 * i)),
            pl.BlockSpec(
                (DIM, TRB),
                lambda i: (0, jnp.minimum(2 * i + 1, VOCAB // TRB)),
            ),
        ],
        out_specs=pl.BlockSpec((TRB, 2 * DIM), lambda i: (i, 0)),
        out_shape=jax.ShapeDtypeStruct((PACK_ROWS, 2 * DIM), jnp.float32),
    )(table_t, table_t)


def _swem_body(idx_hbm, table_hbm, out_hbm, idx_v, rows_a, rows_b, out_v,
               sem_a, sem_b):
    wid = lax.axis_index("s") * NUM_CORES + lax.axis_index("c")
    base = wid * BPW

    pltpu.sync_copy(idx_hbm.at[pl.ds(base, BPW)], idx_v)

    def remap(e, _):
        for j in range(NCHUNK):
            for k in range(CHUNK // LANES):
                v = idx_v[e, j, pl.ds(k * LANES, LANES)]
                r = v & (PAIR - 1)
                adj = jnp.where(r >= TRB, PAIR - 1, 0)
                idx_v[e, j, pl.ds(k * LANES, LANES)] = v + r - adj
        return 0

    lax.fori_loop(0, BPW, remap, 0)

    def fire(e, rows_ref, sem):
        for j in range(NCHUNK):
            pltpu.async_copy(
                table_hbm.at[idx_v.at[e, j, pl.ds(0, CSIZES[j])]],
                rows_ref.at[pl.ds(j * CHUNK, CSIZES[j])],
                sem,
            )

    def drain(rows_ref, sem):
        for j in range(NCHUNK):
            pltpu.make_async_copy(
                table_hbm.at[idx_v.at[0, j, pl.ds(0, CSIZES[j])]],
                rows_ref.at[pl.ds(j * CHUNK, CSIZES[j])],
                sem,
            ).wait()

    def reduce_into(e, rows_ref):
        def body(r, accs):
            return tuple(
                accs[c] + rows_ref[r, pl.ds(c * LANES, LANES)]
                for c in range(NVEC)
            )

        zero = jnp.zeros((LANES,), jnp.float32)
        accs = lax.fori_loop(0, SEQ, body, (zero,) * NVEC, unroll=8)
        for c in range(NVEC):
            out_v[e, pl.ds(c * LANES, LANES)] = accs[c] * (1.0 / SEQ)

    fire(0, rows_a, sem_a)

    def outer(i, _):
        e = 2 * i
        fire(e + 1, rows_b, sem_b)
        drain(rows_a, sem_a)
        reduce_into(e, rows_a)

        @pl.when(e + 2 < BPW)
        def _():
            fire(e + 2, rows_a, sem_a)

        drain(rows_b, sem_b)
        reduce_into(e + 1, rows_b)
        return 0

    lax.fori_loop(0, BPW // 2, outer, 0)

    pltpu.sync_copy(out_v, out_hbm.at[pl.ds(base, BPW)])


def _swem_sc(idx, table):
    mesh = plsc.VectorSubcoreMesh(
        core_axis_name="c",
        subcore_axis_name="s",
        num_cores=NUM_CORES,
        num_subcores=NUM_SUBCORES,
    )
    k = pl.kernel(
        _swem_body,
        out_type=jax.ShapeDtypeStruct((BATCH, DIM), jnp.float32),
        mesh=mesh,
        scratch_types=[
            pltpu.VMEM((BPW, NCHUNK, CHUNK), jnp.int32),
            pltpu.VMEM((ROWS, DIM), jnp.float32),
            pltpu.VMEM((ROWS, DIM), jnp.float32),
            pltpu.VMEM((BPW, DIM), jnp.float32),
            pltpu.SemaphoreType.DMA,
            pltpu.SemaphoreType.DMA,
        ],
        compiler_params=pltpu.CompilerParams(use_tc_tiling_on_sc=False),
    )
    return k(idx, table)


@jax.jit
def _swem(input, table):
    packed = _to_row_major(table.T)
    table_sc = packed.reshape(SC_ROWS, DIM)
    idx3 = jnp.pad(input, ((0, 0), (0, NCHUNK * CHUNK - SEQ))).reshape(
        BATCH, NCHUNK, CHUNK
    )
    return _swem_sc(idx3, table_sc)


def kernel(input, table):
    return _swem(input, table)


# speedup vs baseline: 6.0624x; 1.0912x over previous
import functools

import jax
import jax.numpy as jnp
from jax import lax
from jax.experimental import pallas as pl
from jax.experimental.pallas import tpu as pltpu
from jax.experimental.pallas import tpu_sc as plsc

VOCAB = 1000000
BATCH = 4096
SEQ = 200
DIM = 64
NUM_CORES = 2
NUM_SUBCORES = 16
NUM_WORKERS = NUM_CORES * NUM_SUBCORES
BPW = BATCH // NUM_WORKERS
CHUNK = 96
TAIL = SEQ - 2 * CHUNK
NCHUNK = 3
CSIZES = (CHUNK, CHUNK, TAIL)
ROWS = SEQ
LANES = 16
NVEC = DIM // LANES

TRB = 8192
PAIR = 2 * TRB
NBLK = (VOCAB + PAIR - 1) // PAIR
PACK_ROWS = NBLK * TRB
SC_ROWS = 2 * PACK_ROWS


def _pack_body(x1_ref, x2_ref, o_ref):
    o_ref[...] = jnp.concatenate([x1_ref[...].T, x2_ref[...].T], axis=1)


def _to_row_major(table_t):
    return pl.pallas_call(
        _pack_body,
        grid=(NBLK,),
        in_specs=[
            pl.BlockSpec((DIM, TRB), lambda i: (0, 2 * i)),
            pl.BlockSpec(
                (DIM, TRB),
                lambda i: (0, jnp.minimum(2 * i + 1, VOCAB // TRB)),
            ),
        ],
        out_specs=pl.BlockSpec((TRB, 2 * DIM), lambda i: (i, 0)),
        out_shape=jax.ShapeDtypeStruct((PACK_ROWS, 2 * DIM), jnp.float32),
    )(table_t, table_t)


def _swem_body(idx_hbm, table_hbm, out_hbm, idx_v, rows_a, rows_b, out_v,
               sem_a, sem_b):
    wid = lax.axis_index("s") * NUM_CORES + lax.axis_index("c")
    base = wid * BPW

    pltpu.sync_copy(idx_hbm.at[pl.ds(base, BPW)], idx_v)

    def remap(e, _):
        for j in range(NCHUNK):
            for k in range(CHUNK // LANES):
                v = idx_v[e, j, pl.ds(k * LANES, LANES)]
                r = v & (PAIR - 1)
                adj = jnp.where(r >= TRB, PAIR - 1, 0)
                idx_v[e, j, pl.ds(k * LANES, LANES)] = v + r - adj
        return 0

    lax.fori_loop(0, BPW, remap, 0)

    def fire(e, rows_ref, sem):
        for j in range(NCHUNK):
            pltpu.async_copy(
                table_hbm.at[idx_v.at[e, j, pl.ds(0, CSIZES[j])]],
                rows_ref.at[pl.ds(j * CHUNK, CSIZES[j])],
                sem,
            )

    def drain(rows_ref, sem):
        for j in range(NCHUNK):
            pltpu.make_async_copy(
                table_hbm.at[idx_v.at[0, j, pl.ds(0, CSIZES[j])]],
                rows_ref.at[pl.ds(j * CHUNK, CSIZES[j])],
                sem,
            ).wait()

    def reduce_into(e, rows_ref):
        def body(r, accs):
            return tuple(
                accs[c] + rows_ref[r, pl.ds(c * LANES, LANES)]
                for c in range(NVEC)
            )

        zero = jnp.zeros((LANES,), jnp.float32)
        accs = lax.fori_loop(0, SEQ, body, (zero,) * NVEC, unroll=8)
        for c in range(NVEC):
            out_v[e, pl.ds(c * LANES, LANES)] = accs[c] * (1.0 / SEQ)

    fire(0, rows_a, sem_a)

    def outer(i, _):
        e = 2 * i
        fire(e + 1, rows_b, sem_b)
        drain(rows_a, sem_a)
        reduce_into(e, rows_a)

        @pl.when(e + 2 < BPW)
        def _():
            fire(e + 2, rows_a, sem_a)

        drain(rows_b, sem_b)
        reduce_into(e + 1, rows_b)
        return 0

    lax.fori_loop(0, BPW // 2, outer, 0)

    pltpu.sync_copy(out_v, out_hbm.at[pl.ds(base, BPW)])


def _swem_sc(idx, table):
    mesh = plsc.VectorSubcoreMesh(
        core_axis_name="c",
        subcore_axis_name="s",
        num_cores=NUM_CORES,
        num_subcores=NUM_SUBCORES,
    )
    k = pl.kernel(
        _swem_body,
        out_type=jax.ShapeDtypeStruct((BATCH, DIM), jnp.float32),
        mesh=mesh,
        scratch_types=[
            pltpu.VMEM((BPW, NCHUNK, CHUNK), jnp.int32),
            pltpu.VMEM((ROWS, DIM), jnp.float32),
            pltpu.VMEM((ROWS, DIM), jnp.float32),
            pltpu.VMEM((BPW, DIM), jnp.float32),
            pltpu.SemaphoreType.DMA,
            pltpu.SemaphoreType.DMA,
        ],
        compiler_params=pltpu.CompilerParams(use_tc_tiling_on_sc=False),
    )
    return k(idx, table)


@jax.jit
def _swem(input, table):
    packed = _to_row_major(table.T)
    table_sc = packed.reshape(SC_ROWS, DIM)
    idx3 = jnp.pad(input, ((0, 0), (0, NCHUNK * CHUNK - SEQ))).reshape(
        BATCH, NCHUNK, CHUNK
    )
    return _swem_sc(idx3, table_sc)


def kernel(input, table):
    return _swem(input, table)


# speedup vs baseline: 6.2735x; 1.0348x over previous
import functools

import jax
import jax.numpy as jnp
from jax import lax
from jax.experimental import pallas as pl
from jax.experimental.pallas import tpu as pltpu
from jax.experimental.pallas import tpu_sc as plsc

VOCAB = 1000000
BATCH = 4096
SEQ = 200
DIM = 64
NUM_CORES = 2
NUM_SUBCORES = 16
NUM_WORKERS = NUM_CORES * NUM_SUBCORES
BPW = BATCH // NUM_WORKERS
CHUNK = 96
TAIL = SEQ - 2 * CHUNK
NCHUNK = 3
CSIZES = (CHUNK, CHUNK, TAIL)
ROWS = SEQ
LANES = 16
NVEC = DIM // LANES

TRB = 16384
PAIR = 2 * TRB
NBLK = (VOCAB + PAIR - 1) // PAIR
PACK_ROWS = NBLK * TRB
SC_ROWS = 2 * PACK_ROWS


def _pack_body(x1_ref, x2_ref, o_ref):
    o_ref[...] = jnp.concatenate([x1_ref[...].T, x2_ref[...].T], axis=1)


def _to_row_major(table_t):
    return pl.pallas_call(
        _pack_body,
        grid=(NBLK,),
        in_specs=[
            pl.BlockSpec((DIM, TRB), lambda i: (0, 2 * i)),
            pl.BlockSpec(
                (DIM, TRB),
                lambda i: (0, jnp.minimum(2 * i + 1, VOCAB // TRB)),
            ),
        ],
        out_specs=pl.BlockSpec((TRB, 2 * DIM), lambda i: (i, 0)),
        out_shape=jax.ShapeDtypeStruct((PACK_ROWS, 2 * DIM), jnp.float32),
    )(table_t, table_t)


def _swem_body(idx_hbm, table_hbm, out_hbm, idx_v, rows_a, rows_b, out_v,
               sem_a, sem_b):
    wid = lax.axis_index("s") * NUM_CORES + lax.axis_index("c")
    base = wid * BPW

    pltpu.sync_copy(idx_hbm.at[pl.ds(base, BPW)], idx_v)

    def remap(e, _):
        for j in range(NCHUNK):
            for k in range(CHUNK // LANES):
                v = idx_v[e, j, pl.ds(k * LANES, LANES)]
                r = v & (PAIR - 1)
                adj = jnp.where(r >= TRB, PAIR - 1, 0)
                idx_v[e, j, pl.ds(k * LANES, LANES)] = v + r - adj
        return 0

    lax.fori_loop(0, BPW, remap, 0)

    def fire(e, rows_ref, sem):
        for j in range(NCHUNK):
            pltpu.async_copy(
                table_hbm.at[idx_v.at[e, j, pl.ds(0, CSIZES[j])]],
                rows_ref.at[pl.ds(j * CHUNK, CSIZES[j])],
                sem,
            )

    def drain(rows_ref, sem):
        for j in range(NCHUNK):
            pltpu.make_async_copy(
                table_hbm.at[idx_v.at[0, j, pl.ds(0, CSIZES[j])]],
                rows_ref.at[pl.ds(j * CHUNK, CSIZES[j])],
                sem,
            ).wait()

    def reduce_into(e, rows_ref):
        def body(r, accs):
            return tuple(
                accs[c] + rows_ref[r, pl.ds(c * LANES, LANES)]
                for c in range(NVEC)
            )

        zero = jnp.zeros((LANES,), jnp.float32)
        accs = lax.fori_loop(0, SEQ, body, (zero,) * NVEC, unroll=8)
        for c in range(NVEC):
            out_v[e, pl.ds(c * LANES, LANES)] = accs[c] * (1.0 / SEQ)

    fire(0, rows_a, sem_a)

    def outer(i, _):
        e = 2 * i
        fire(e + 1, rows_b, sem_b)
        drain(rows_a, sem_a)
        reduce_into(e, rows_a)

        @pl.when(e + 2 < BPW)
        def _():
            fire(e + 2, rows_a, sem_a)

        drain(rows_b, sem_b)
        reduce_into(e + 1, rows_b)
        return 0

    lax.fori_loop(0, BPW // 2, outer, 0)

    pltpu.sync_copy(out_v, out_hbm.at[pl.ds(base, BPW)])


def _swem_sc(idx, table):
    mesh = plsc.VectorSubcoreMesh(
        core_axis_name="c",
        subcore_axis_name="s",
        num_cores=NUM_CORES,
        num_subcores=NUM_SUBCORES,
    )
    k = pl.kernel(
        _swem_body,
        out_type=jax.ShapeDtypeStruct((BATCH, DIM), jnp.float32),
        mesh=mesh,
        scratch_types=[
            pltpu.VMEM((BPW, NCHUNK, CHUNK), jnp.int32),
            pltpu.VMEM((ROWS, DIM), jnp.float32),
            pltpu.VMEM((ROWS, DIM), jnp.float32),
            pltpu.VMEM((BPW, DIM), jnp.float32),
            pltpu.SemaphoreType.DMA,
            pltpu.SemaphoreType.DMA,
        ],
        compiler_params=pltpu.CompilerParams(use_tc_tiling_on_sc=False),
    )
    return k(idx, table)


@jax.jit
def _swem(input, table):
    packed = _to_row_major(table.T)
    table_sc = packed.reshape(SC_ROWS, DIM)
    idx3 = jnp.pad(input, ((0, 0), (0, NCHUNK * CHUNK - SEQ))).reshape(
        BATCH, NCHUNK, CHUNK
    )
    return _swem_sc(idx3, table_sc)


def kernel(input, table):
    return _swem(input, table)


# speedup vs baseline: 7.2755x; 1.1597x over previous
import functools

import jax
import jax.numpy as jnp
from jax import lax
from jax.experimental import pallas as pl
from jax.experimental.pallas import tpu as pltpu
from jax.experimental.pallas import tpu_sc as plsc

VOCAB = 1000000
BATCH = 4096
SEQ = 200
DIM = 64
NUM_CORES = 2
NUM_SUBCORES = 16
NUM_WORKERS = NUM_CORES * NUM_SUBCORES
BPW = BATCH // NUM_WORKERS
CHUNK = 96
TAIL = SEQ - 2 * CHUNK
NCHUNK = 3
CSIZES = (CHUNK, CHUNK, TAIL)
ROWS = SEQ
LANES = 16
NVEC = DIM // LANES

TRB = 16384
PAIR = 2 * TRB
NBLK = (VOCAB + PAIR - 1) // PAIR
PACK_ROWS = NBLK * TRB
SC_ROWS = 2 * PACK_ROWS


def _pack_body(x1_ref, x2_ref, o_ref):
    o_ref[...] = jnp.concatenate([x1_ref[...], x2_ref[...]], axis=0).T


def _to_row_major(table_t):
    return pl.pallas_call(
        _pack_body,
        grid=(NBLK,),
        in_specs=[
            pl.BlockSpec((DIM, TRB), lambda i: (0, 2 * i)),
            pl.BlockSpec(
                (DIM, TRB),
                lambda i: (0, jnp.minimum(2 * i + 1, VOCAB // TRB)),
            ),
        ],
        out_specs=pl.BlockSpec((TRB, 2 * DIM), lambda i: (i, 0)),
        out_shape=jax.ShapeDtypeStruct((PACK_ROWS, 2 * DIM), jnp.float32),
    )(table_t, table_t)


def _swem_body(idx_hbm, table_hbm, out_hbm, idx_v, rows_a, rows_b, out_v,
               sem_a, sem_b):
    wid = lax.axis_index("s") * NUM_CORES + lax.axis_index("c")
    base = wid * BPW

    pltpu.sync_copy(idx_hbm.at[pl.ds(base, BPW)], idx_v)

    def remap(e, _):
        for j in range(NCHUNK):
            for k in range(CHUNK // LANES):
                v = idx_v[e, j, pl.ds(k * LANES, LANES)]
                r = v & (PAIR - 1)
                adj = jnp.where(r >= TRB, PAIR - 1, 0)
                idx_v[e, j, pl.ds(k * LANES, LANES)] = v + r - adj
        return 0

    lax.fori_loop(0, BPW, remap, 0)

    def fire(e, rows_ref, sem):
        for j in range(NCHUNK):
            pltpu.async_copy(
                table_hbm.at[idx_v.at[e, j, pl.ds(0, CSIZES[j])]],
                rows_ref.at[pl.ds(j * CHUNK, CSIZES[j])],
                sem,
            )

    def drain(rows_ref, sem):
        for j in range(NCHUNK):
            pltpu.make_async_copy(
                table_hbm.at[idx_v.at[0, j, pl.ds(0, CSIZES[j])]],
                rows_ref.at[pl.ds(j * CHUNK, CSIZES[j])],
                sem,
            ).wait()

    def reduce_into(e, rows_ref):
        def body(r, accs):
            return tuple(
                accs[c] + rows_ref[r, pl.ds(c * LANES, LANES)]
                for c in range(NVEC)
            )

        zero = jnp.zeros((LANES,), jnp.float32)
        accs = lax.fori_loop(0, SEQ, body, (zero,) * NVEC, unroll=8)
        for c in range(NVEC):
            out_v[e, pl.ds(c * LANES, LANES)] = accs[c] * (1.0 / SEQ)

    fire(0, rows_a, sem_a)

    def outer(i, _):
        e = 2 * i
        fire(e + 1, rows_b, sem_b)
        drain(rows_a, sem_a)
        reduce_into(e, rows_a)

        @pl.when(e + 2 < BPW)
        def _():
            fire(e + 2, rows_a, sem_a)

        drain(rows_b, sem_b)
        reduce_into(e + 1, rows_b)
        return 0

    lax.fori_loop(0, BPW // 2, outer, 0)

    pltpu.sync_copy(out_v, out_hbm.at[pl.ds(base, BPW)])


def _swem_sc(idx, table):
    mesh = plsc.VectorSubcoreMesh(
        core_axis_name="c",
        subcore_axis_name="s",
        num_cores=NUM_CORES,
        num_subcores=NUM_SUBCORES,
    )
    k = pl.kernel(
        _swem_body,
        out_type=jax.ShapeDtypeStruct((BATCH, DIM), jnp.float32),
        mesh=mesh,
        scratch_types=[
            pltpu.VMEM((BPW, NCHUNK, CHUNK), jnp.int32),
            pltpu.VMEM((ROWS, DIM), jnp.float32),
            pltpu.VMEM((ROWS, DIM), jnp.float32),
            pltpu.VMEM((BPW, DIM), jnp.float32),
            pltpu.SemaphoreType.DMA,
            pltpu.SemaphoreType.DMA,
        ],
        compiler_params=pltpu.CompilerParams(use_tc_tiling_on_sc=False),
    )
    return k(idx, table)


@jax.jit
def _swem(input, table):
    packed = _to_row_major(table.T)
    table_sc = packed.reshape(SC_ROWS, DIM)
    idx3 = jnp.pad(input, ((0, 0), (0, NCHUNK * CHUNK - SEQ))).reshape(
        BATCH, NCHUNK, CHUNK
    )
    return _swem_sc(idx3, table_sc)


def kernel(input, table):
    return _swem(input, table)
